# baseline (speedup 1.0000x reference)
import jax
import jax.numpy as jnp
from jax import lax
from jax.experimental import pallas as pl
from jax.experimental.pallas import tpu as pltpu
from jax.experimental.pallas import tpu_sc as plsc

_H = 128
_T = 200
_NW = 32
_ROWS_PER_W = 32
_HV = _H // 16
_NPW = _ROWS_PER_W * _T
_T1 = 128
_T2 = _T - _T1
_BR = 25


def _emb_body(tok_hbm, seg_hbm, table_hbm, segtab_hbm, pos_hbm, out_hbm,
              tokall_v, segall_v, idx2a_v, idx2b_v, idx2c_v, idx2d_v, rows_v,
              segtab_v, posab_sp, asem, gsem, ssem):
    idx2 = (idx2a_v, idx2b_v, idx2c_v, idx2d_v)
    c = lax.axis_index("c")
    s = lax.axis_index("s")
    wid = s * 2 + c
    base0 = wid * _NPW

    pltpu.sync_copy(tok_hbm.at[pl.ds(base0, _NPW)], tokall_v.at[pl.ds(0, _NPW)])
    pltpu.sync_copy(seg_hbm.at[pl.ds(base0, _NPW)], segall_v.at[pl.ds(0, _NPW)])

    pltpu.sync_copy(segtab_hbm, segtab_v)

    def _build_block(blk):
        half = blk // _BR
        p0 = pl.multiple_of((blk % _BR) * 8, 8)
        hf = lax.broadcast_in_dim(half, (16,), ()).astype(jnp.float32)
        pltpu.sync_copy(pos_hbm.at[pl.ds(p0, 8)], rows_v.at[1, pl.ds(0, 8)])
        for r in range(8):
            for h in range(_HV):
                sl = pl.ds(h * 16, 16)
                s0 = segtab_v[0, sl]
                segv = s0 + hf * (segtab_v[1, sl] - s0)
                rows_v[0, r, sl] = rows_v[1, r, sl] + segv
        pltpu.sync_copy(rows_v.at[0, pl.ds(0, 8)],
                        posab_sp.at[pl.ds(pl.multiple_of(blk * 8, 8), 8)])

    for k in range(4):
        blk = s + 16 * k

        @pl.when(blk < 2 * _BR)
        def _do_build(blk=blk):
            _build_block(blk)

    plsc.subcore_barrier()

    def _mk_idx2(j, b):
        for g in range(13):
            sv = segall_v[pl.ds(j * _T + g * 16, 16)]
            iv = sv * _T + (g * 16 + lax.iota(jnp.int32, 16))
            idx2[b][pl.ds(g * 16, 16)] = iv

    def _issue_a(j, b):
        pltpu.async_copy(posab_sp.at[idx2[b].at[pl.ds(0, _T1)]],
                         rows_v.at[b, pl.ds(0, _T1)], asem)
        pltpu.async_copy(posab_sp.at[idx2[b].at[pl.ds(_T1, _T2)]],
                         rows_v.at[b, pl.ds(_T1, _T2)], asem)

    def _issue_g(j, b):
        base = j * _T
        pltpu.async_copy(table_hbm.at[tokall_v.at[pl.ds(base, _T1)]],
                         rows_v.at[b, pl.ds(0, _T1)], gsem, add=True)
        pltpu.async_copy(table_hbm.at[tokall_v.at[pl.ds(base + _T1, _T2)]],
                         rows_v.at[b, pl.ds(_T1, _T2)], gsem, add=True)

    def _issue_s(j, b):
        pltpu.async_copy(rows_v.at[b], out_hbm.at[pl.ds(base0 + j * _T, _T)],
                         ssem)

    def _drain(sem, b):
        pltpu.make_async_copy(out_hbm.at[pl.ds(0, _T)], rows_v.at[b], sem).wait()

    def _drain_s(b):
        pltpu.make_async_copy(rows_v.at[b], out_hbm.at[pl.ds(0, _T)], ssem).wait()

    _mk_idx2(0, 0)
    _issue_a(0, 0)
    _mk_idx2(1, 1)
    _issue_a(1, 1)

    for j in range(_ROWS_PER_W):
        b = j % 4
        _drain(asem, b)
        _issue_g(j, b)
        if j >= 2:
            _drain_s((j + 2) % 4)
        if j + 2 < _ROWS_PER_W:
            _mk_idx2(j + 2, (j + 2) % 4)
            _issue_a(j + 2, (j + 2) % 4)
        _drain(gsem, b)
        _issue_s(j, b)
    _drain_s(30 % 4)
    _drain_s(31 % 4)


def _emb(tok, seg, token_table, segment_table, pos_weight):
    n = tok.shape[0]
    mesh = plsc.VectorSubcoreMesh(core_axis_name="c", subcore_axis_name="s")
    return pl.kernel(
        _emb_body,
        out_type=jax.ShapeDtypeStruct((n, _H), jnp.float32),
        mesh=mesh,
        scratch_types=[
            pltpu.VMEM((_NPW + 16,), jnp.int32),
            pltpu.VMEM((_NPW + 16,), jnp.int32),
            pltpu.VMEM((208,), jnp.int32),
            pltpu.VMEM((208,), jnp.int32),
            pltpu.VMEM((208,), jnp.int32),
            pltpu.VMEM((208,), jnp.int32),
            pltpu.VMEM((4, _T, _H), jnp.float32),
            pltpu.VMEM((2, _H), jnp.float32),
            pltpu.VMEM_SHARED((2 * _T, _H), jnp.float32),
            pltpu.SemaphoreType.DMA,
            pltpu.SemaphoreType.DMA,
            pltpu.SemaphoreType.DMA,
        ],
    )(tok, seg, token_table, segment_table, pos_weight)


def kernel(tokens, segments, token_table, segment_table, pos_weight):
    b, t = tokens.shape
    h = token_table.shape[1]
    tok = tokens.reshape(b * t).astype(jnp.int32)
    seg = segments.reshape(b * t).astype(jnp.int32)
    out = _emb(tok, seg, token_table, segment_table, pos_weight)
    return out.reshape(b, t, h)
